# baseline (speedup 1.0000x reference)
import functools

import jax
import jax.numpy as jnp
from jax.experimental import pallas as pl
from jax.experimental.pallas import tpu as pltpu


_VMEM_LIMIT = 48 * 1024 * 1024
_EPS = 1e-5


def _l1_kernel(a_ref, w_ref, o_ref, s_ref, sq_ref):
    A = a_ref[...]
    taps = [A[:, qh:qh + 126, qw:qw + 126, :]
            for qh in range(2) for qw in range(2)]
    X = jnp.concatenate(taps, axis=-1).reshape(126 * 126, 48)
    out = jnp.dot(X, w_ref[...], preferred_element_type=jnp.float32)
    s_ref[...] = jnp.sum(out, axis=0, keepdims=True)[None]
    sq_ref[...] = jnp.sum(out * out, axis=0, keepdims=True)[None]
    o_ref[...] = out.reshape(1, 126, 126, 64).astype(o_ref.dtype)


def _layer1(x, w_mat):
    N = x.shape[0]
    xs2d = (x.reshape(N, 3, 127, 2, 127, 2)
            .transpose(0, 2, 4, 3, 5, 1)
            .reshape(N, 127, 127, 12)
            .astype(jnp.bfloat16))
    w1 = (w_mat.reshape(3, 2, 2, 2, 2, 64)
          .transpose(1, 3, 2, 4, 0, 5)
          .reshape(48, 64))
    pre, s, sq = pl.pallas_call(
        _l1_kernel,
        out_shape=(jax.ShapeDtypeStruct((N, 126, 126, 64), jnp.float32),
                   jax.ShapeDtypeStruct((N, 1, 64), jnp.float32),
                   jax.ShapeDtypeStruct((N, 1, 64), jnp.float32)),
        grid_spec=pltpu.PrefetchScalarGridSpec(
            num_scalar_prefetch=0,
            grid=(N,),
            in_specs=[pl.BlockSpec((1, 127, 127, 12), lambda g: (g, 0, 0, 0)),
                      pl.BlockSpec((48, 64), lambda g: (0, 0))],
            out_specs=(pl.BlockSpec((1, 126, 126, 64), lambda g: (g, 0, 0, 0)),
                       pl.BlockSpec((1, 1, 64), lambda g: (g, 0, 0)),
                       pl.BlockSpec((1, 1, 64), lambda g: (g, 0, 0))),
        ),
        compiler_params=pltpu.CompilerParams(
            dimension_semantics=("parallel",),
            vmem_limit_bytes=_VMEM_LIMIT),
    )(xs2d, w1)
    return pre, s, sq


def _conv_kernel(a_ref, w_ref, scale_ref, shift_ref, o_ref, s_ref, sq_ref, *,
                 OH, OW, Cin):
    A = a_ref[...]
    G, H, W2 = A.shape[0], A.shape[1], A.shape[2]
    y = (A.astype(jnp.float32) * scale_ref[...].reshape(1, 1, 1, 2 * Cin)
         + shift_ref[...].reshape(1, 1, 1, 2 * Cin))
    a = jnp.where(y >= 0.0, y, 0.2 * y).astype(jnp.bfloat16)
    a5 = a.reshape(G, H // 2, 2, W2, 2 * Cin)
    taps = []
    for r in range(4):
        q, p = divmod(r, 2)
        ar = a5[:, q:q + OH, p]
        for cq in range(2):
            taps.append(ar[:, :, cq:cq + OW, :])
    X = jnp.concatenate(taps, axis=-1).reshape(G * OH * OW, 16 * Cin)
    out = jnp.dot(X, w_ref[...], preferred_element_type=jnp.float32)
    s_ref[...] = jnp.sum(out, axis=0, keepdims=True)[None]
    sq_ref[...] = jnp.sum(out * out, axis=0, keepdims=True)[None]
    o_ref[...] = out.reshape(G, OH, OW, -1).astype(o_ref.dtype)


def _conv_layer(pre_in, w_mat, scale, shift, G, ns):
    N, H, W, Cin = pre_in.shape
    Cout = w_mat.shape[1]
    OH, OW = H // 2 - 1, W // 2 - 1
    ng = N // G
    Cb = Cout // ns
    W2 = W // 2
    a_packed = pre_in.reshape(N, H, W2, 2 * Cin)
    scale2 = jnp.tile(scale, 2).reshape(1, 2 * Cin)
    shift2 = jnp.tile(shift, 2).reshape(1, 2 * Cin)
    w2 = (w_mat.reshape(Cin, 4, 2, 2, Cout)
          .transpose(1, 2, 3, 0, 4)
          .reshape(16 * Cin, Cout))
    kern = functools.partial(_conv_kernel, OH=OH, OW=OW, Cin=Cin)
    pre, s, sq = pl.pallas_call(
        kern,
        out_shape=(jax.ShapeDtypeStruct((N, OH, OW, Cout), jnp.float32),
                   jax.ShapeDtypeStruct((ng, 1, Cout), jnp.float32),
                   jax.ShapeDtypeStruct((ng, 1, Cout), jnp.float32)),
        grid_spec=pltpu.PrefetchScalarGridSpec(
            num_scalar_prefetch=0,
            grid=(ng, ns),
            in_specs=[pl.BlockSpec((G, H, W2, 2 * Cin),
                                   lambda g, j: (g, 0, 0, 0)),
                      pl.BlockSpec((16 * Cin, Cb), lambda g, j: (0, j)),
                      pl.BlockSpec((1, 2 * Cin), lambda g, j: (0, 0)),
                      pl.BlockSpec((1, 2 * Cin), lambda g, j: (0, 0))],
            out_specs=(pl.BlockSpec((G, OH, OW, Cb),
                                    lambda g, j: (g, 0, 0, j)),
                       pl.BlockSpec((1, 1, Cb), lambda g, j: (g, 0, j)),
                       pl.BlockSpec((1, 1, Cb), lambda g, j: (g, 0, j))),
        ),
        compiler_params=pltpu.CompilerParams(
            dimension_semantics=("parallel", "parallel"),
            vmem_limit_bytes=_VMEM_LIMIT),
    )(a_packed, w2, scale2, shift2)
    return pre, s, sq


def _bn_coeffs(s_part, sq_part, gamma, beta, M):
    s = jnp.sum(s_part, axis=(0, 1))
    sq = jnp.sum(sq_part, axis=(0, 1))
    inv_m = 1.0 / M
    mean = s * inv_m
    var = jnp.maximum(sq * inv_m - mean * mean, 0.0)
    scale = gamma * jax.lax.rsqrt(var + _EPS)
    shift = beta - mean * scale
    return scale, shift


def _tail_kernel(x_ref, sc_ref, sh_ref, w_ref, b_ref, o_ref):
    y = x_ref[...].astype(jnp.float32) * sc_ref[...] + sh_ref[...]
    a = jnp.where(y >= 0.0, y, 0.2 * y)
    prod = a * w_ref[...]
    t = jnp.sum(prod, axis=2, keepdims=True)
    t = jnp.sum(t, axis=1, keepdims=True) + b_ref[...]
    o_ref[...] = 1.0 / (1.0 + jnp.exp(-t))


def _tail(pre5, scale5, shift5, tail_w, tail_b):
    N = pre5.shape[0]
    wt = tail_w.reshape(3, 3, 1024, 4, 4)
    w_eff = jnp.zeros((6, 6, 1024), jnp.float32)
    for oh in range(3):
        for ow in range(3):
            w_eff = w_eff.at[oh:oh + 4, ow:ow + 4, :].add(
                jnp.transpose(wt[oh, ow], (1, 2, 0)))
    out = pl.pallas_call(
        _tail_kernel,
        out_shape=jax.ShapeDtypeStruct((N, 1, 1), jnp.float32),
        compiler_params=pltpu.CompilerParams(vmem_limit_bytes=_VMEM_LIMIT),
    )(pre5.reshape(N, 36, 1024),
      scale5.reshape(1, 1, 1024),
      shift5.reshape(1, 1, 1024),
      w_eff.reshape(1, 36, 1024),
      tail_b.reshape(1, 1, 1))
    return out.reshape(N, 1)


def kernel(x, conv1_w_mat, bn1_gamma, bn1_beta, conv2_w_mat, bn2_gamma,
           bn2_beta, conv3_w_mat, bn3_gamma, bn3_beta, conv4_w_mat, bn4_gamma,
           bn4_beta, conv5_w_mat, bn5_gamma, bn5_beta, tail_w, tail_b):
    N = x.shape[0]
    pre, s, sq = _layer1(x, conv1_w_mat)
    scale, shift = _bn_coeffs(s, sq, bn1_gamma, bn1_beta, N * 126 * 126)

    layer_cfg = [(conv2_w_mat, bn2_gamma, bn2_beta, 1, 1),
                 (conv3_w_mat, bn3_gamma, bn3_beta, 2, 1),
                 (conv4_w_mat, bn4_gamma, bn4_beta, 4, 2),
                 (conv5_w_mat, bn5_gamma, bn5_beta, 8, 4)]
    for w_mat, gamma, beta, G, ns in layer_cfg:
        pre, s, sq = _conv_layer(pre, w_mat, scale, shift, G, ns)
        M = pre.shape[0] * pre.shape[1] * pre.shape[2]
        scale, shift = _bn_coeffs(s, sq, gamma, beta, M)

    return _tail(pre, scale, shift, tail_w, tail_b)
